# baseline (speedup 1.0000x reference)
import jax
import jax.numpy as jnp
from jax.experimental import pallas as pl

EPS = 1e-09
N = 10000
BM = 1024
BN = 1280


def _gram_body(a_ref, b_ref, o_ref):
    a = a_ref[...]
    b = b_ref[...]
    o_ref[...] = jax.lax.dot_general(
        a, b, (((1,), (1,)), ((), ())), preferred_element_type=jnp.float32)


def _gram(h):
    n = h.shape[0]
    grid = (pl.cdiv(n, BM), pl.cdiv(n, BN))
    return pl.pallas_call(
        _gram_body,
        grid=grid,
        in_specs=[
            pl.BlockSpec((BM, h.shape[1]), lambda i, j: (i, 0)),
            pl.BlockSpec((BN, h.shape[1]), lambda i, j: (j, 0)),
        ],
        out_specs=pl.BlockSpec((BM, BN), lambda i, j: (i, j)),
        out_shape=jax.ShapeDtypeStruct((n, n), jnp.float32),
    )(h, h)


def kernel(x, edge_index, edge_attr, emb, lin_edge1_w, lin_edge1_b, nn1_w, nn1_b,
           lin_edge2_w, lin_edge2_b, nn2_w, nn2_b):
    h = jnp.take(emb, x[:, 0], axis=0)
    src, dst = edge_index[0], edge_index[1]

    e = edge_attr @ lin_edge1_w + lin_edge1_b
    m = jax.nn.relu(h[src] + e)
    aggr = jax.ops.segment_sum(m, dst, num_segments=N)
    h = (aggr + (1.0 + EPS) * h) @ nn1_w + nn1_b
    h = jax.nn.leaky_relu(h, negative_slope=0.01)

    e = edge_attr @ lin_edge2_w + lin_edge2_b
    m = jax.nn.relu(h[src] + e)
    aggr = jax.ops.segment_sum(m, dst, num_segments=N)
    h = (aggr + (1.0 + EPS) * h) @ nn2_w + nn2_b

    return _gram(h)


# speedup vs baseline: 1.9265x; 1.9265x over previous
import functools

import jax
import jax.numpy as jnp
from jax import lax
from jax.experimental import pallas as pl
from jax.experimental.pallas import tpu as pltpu
from jax.experimental.pallas import tpu_sc as plsc

EPS = 1e-09
N = 10000
E = 640000
NW = 32
CH = 80
BM = 1024
BN = 1280


def _sc_gather(table, idx):
    e = idx.shape[0]
    per_w = e // NW
    nch = per_w // CH
    idx3 = idx.reshape(NW, nch, CH)
    mesh = plsc.VectorSubcoreMesh(core_axis_name="c", subcore_axis_name="s")

    @functools.partial(
        pl.kernel,
        out_type=jax.ShapeDtypeStruct((e,) + table.shape[1:], table.dtype),
        mesh=mesh,
        compiler_params=pltpu.CompilerParams(use_tc_tiling_on_sc=False),
        scratch_types=[
            pltpu.VMEM((nch, CH), jnp.int32),
            pltpu.VMEM((CH,) + table.shape[1:], table.dtype),
            pltpu.SemaphoreType.DMA,
        ],
    )
    def k(table_hbm, idx_hbm, out_hbm, idx_v, buf_v, sem):
        wid = lax.axis_index("s") * 2 + lax.axis_index("c")
        row0 = wid * nch
        pltpu.sync_copy(idx_hbm.at[wid], idx_v)

        def body(j, carry):
            pltpu.async_copy(table_hbm.at[idx_v.at[j]], buf_v, sem).wait()
            pltpu.sync_copy(buf_v, out_hbm.at[pl.ds((row0 + j) * CH, CH)])
            return carry

        lax.fori_loop(0, nch, body, 0)

    return k(table, idx3)


def _gram_body(a_ref, b_ref, o_ref):
    o_ref[...] = jax.lax.dot_general(
        a_ref[...], b_ref[...], (((1,), (1,)), ((), ())),
        preferred_element_type=jnp.float32)


def _gram(h):
    n = h.shape[0]
    grid = (pl.cdiv(n, BM), pl.cdiv(n, BN))
    return pl.pallas_call(
        _gram_body,
        grid=grid,
        in_specs=[
            pl.BlockSpec((BM, h.shape[1]), lambda i, j: (i, 0)),
            pl.BlockSpec((BN, h.shape[1]), lambda i, j: (j, 0)),
        ],
        out_specs=pl.BlockSpec((BM, BN), lambda i, j: (i, j)),
        out_shape=jax.ShapeDtypeStruct((n, n), jnp.float32),
    )(h, h)


def kernel(x, edge_index, edge_attr, emb, lin_edge1_w, lin_edge1_b, nn1_w, nn1_b,
           lin_edge2_w, lin_edge2_b, nn2_w, nn2_b):
    src, dst = edge_index[0], edge_index[1]

    c = _sc_gather(jnp.broadcast_to(x, (N, 16)), src)[:, 0]

    h = jnp.take(emb, x[:, 0], axis=0)
    onehot = (c[:, None] == jnp.arange(20, dtype=jnp.int32)[None, :]).astype(jnp.float32)
    e = edge_attr @ lin_edge1_w + lin_edge1_b
    m = jax.nn.relu(onehot @ emb + e)
    aggr = jax.ops.segment_sum(m @ nn1_w, dst, num_segments=N)
    h = aggr + (1.0 + EPS) * (h @ nn1_w) + nn1_b
    h = jax.nn.leaky_relu(h, negative_slope=0.01)

    g = _sc_gather(h, src)
    e = edge_attr @ lin_edge2_w + lin_edge2_b
    m = jax.nn.relu(g + e)
    aggr = jax.ops.segment_sum(m @ nn2_w, dst, num_segments=N)
    h = aggr + (1.0 + EPS) * (h @ nn2_w) + nn2_b

    return _gram(h)


# speedup vs baseline: 3.2113x; 1.6669x over previous
import functools

import jax
import jax.numpy as jnp
from jax import lax
from jax.experimental import pallas as pl
from jax.experimental.pallas import tpu as pltpu
from jax.experimental.pallas import tpu_sc as plsc

EPS = 1e-09
N = 10000
E = 640000
NW = 32
CH = 80
BM = 1024
BN = 1280


def _sc_gather(table, idx):
    e = idx.shape[0]
    per_w = e // NW
    nch = per_w // CH
    idx3 = idx.reshape(NW, nch, CH)
    mesh = plsc.VectorSubcoreMesh(core_axis_name="c", subcore_axis_name="s")

    @functools.partial(
        pl.kernel,
        out_type=jax.ShapeDtypeStruct((e,) + table.shape[1:], table.dtype),
        mesh=mesh,
        compiler_params=pltpu.CompilerParams(use_tc_tiling_on_sc=False),
        scratch_types=[
            pltpu.VMEM((nch, CH), jnp.int32),
            pltpu.VMEM((CH,) + table.shape[1:], table.dtype),
            pltpu.SemaphoreType.DMA,
        ],
    )
    def k(table_hbm, idx_hbm, out_hbm, idx_v, buf_v, sem):
        wid = lax.axis_index("s") * 2 + lax.axis_index("c")
        row0 = wid * nch
        pltpu.sync_copy(idx_hbm.at[wid], idx_v)

        def body(j, carry):
            pltpu.async_copy(table_hbm.at[idx_v.at[j]], buf_v, sem).wait()
            pltpu.sync_copy(buf_v, out_hbm.at[pl.ds((row0 + j) * CH, CH)])
            return carry

        lax.fori_loop(0, nch, body, 0)

    return k(table, idx3)


def _sc_scatter_add(vals, idx, n):
    e, d = vals.shape
    per_w = e // NW
    nch = per_w // CH
    idx3 = idx.reshape(NW, nch, CH)
    rpt = n // 16
    zeros = jnp.zeros((n, d), jnp.float32)
    mesh = plsc.VectorSubcoreMesh(core_axis_name="c", subcore_axis_name="s")

    @functools.partial(
        pl.kernel,
        out_type=jax.ShapeDtypeStruct((2, n, d), jnp.float32),
        mesh=mesh,
        compiler_params=pltpu.CompilerParams(use_tc_tiling_on_sc=False),
        scratch_types=[
            pltpu.VMEM((nch, CH), jnp.int32),
            pltpu.VMEM((CH, d), jnp.float32),
            pltpu.VMEM_SHARED((n, d), jnp.float32),
            pltpu.SemaphoreType.DMA,
        ],
    )
    def k(vals_hbm, idx_hbm, zeros_hbm, out_hbm, idx_v, buf_v, acc_sh, sem):
        cid = lax.axis_index("c")
        sid = lax.axis_index("s")
        wid = sid * 2 + cid
        pltpu.sync_copy(zeros_hbm.at[pl.ds(sid * rpt, rpt)],
                        acc_sh.at[pl.ds(sid * rpt, rpt)])
        plsc.subcore_barrier()
        pltpu.sync_copy(idx_hbm.at[wid], idx_v)

        def body(j, carry):
            pltpu.async_copy(
                vals_hbm.at[pl.ds((wid * nch + j) * CH, CH)], buf_v, sem).wait()
            pltpu.sync_copy(buf_v, acc_sh.at[idx_v.at[j]], add=True)
            return carry

        lax.fori_loop(0, nch, body, 0)
        plsc.subcore_barrier()
        pltpu.sync_copy(acc_sh.at[pl.ds(sid * rpt, rpt)],
                        out_hbm.at[cid].at[pl.ds(sid * rpt, rpt)])

    return k(vals, idx3, zeros)


def _gram_body(a_ref, b_ref, o_ref):
    o_ref[...] = jax.lax.dot_general(
        a_ref[...], b_ref[...], (((1,), (1,)), ((), ())),
        preferred_element_type=jnp.float32)


def _gram(h):
    n = h.shape[0]
    grid = (pl.cdiv(n, BM), pl.cdiv(n, BN))
    return pl.pallas_call(
        _gram_body,
        grid=grid,
        in_specs=[
            pl.BlockSpec((BM, h.shape[1]), lambda i, j: (i, 0)),
            pl.BlockSpec((BN, h.shape[1]), lambda i, j: (j, 0)),
        ],
        out_specs=pl.BlockSpec((BM, BN), lambda i, j: (i, j)),
        out_shape=jax.ShapeDtypeStruct((n, n), jnp.float32),
    )(h, h)


def kernel(x, edge_index, edge_attr, emb, lin_edge1_w, lin_edge1_b, nn1_w, nn1_b,
           lin_edge2_w, lin_edge2_b, nn2_w, nn2_b):
    src, dst = edge_index[0], edge_index[1]

    c = _sc_gather(jnp.broadcast_to(x, (N, 16)), src)[:, 0]

    h = jnp.take(emb, x[:, 0], axis=0)
    onehot = (c[:, None] == jnp.arange(20, dtype=jnp.int32)[None, :]).astype(jnp.float32)
    e = edge_attr @ lin_edge1_w + lin_edge1_b
    m = jax.nn.relu(onehot @ emb + e)
    part = _sc_scatter_add(m @ nn1_w, dst, N)
    h = part[0] + part[1] + (1.0 + EPS) * (h @ nn1_w) + nn1_b
    h = jax.nn.leaky_relu(h, negative_slope=0.01)

    g = _sc_gather(h, src)
    e = edge_attr @ lin_edge2_w + lin_edge2_b
    m = jax.nn.relu(g + e)
    part = _sc_scatter_add(m @ nn2_w, dst, N)
    h = part[0] + part[1] + (1.0 + EPS) * (h @ nn2_w) + nn2_b

    return _gram(h)
